# Initial kernel scaffold; baseline (speedup 1.0000x reference)
import jax
import jax.numpy as jnp
from jax.experimental import pallas as pl


def kernel(x):
    raise NotImplementedError("write your pallas kernel here")


# speedup vs baseline: 3.3221x; 3.3221x over previous
import functools

import jax
import jax.numpy as jnp
from jax import lax
from jax.experimental import pallas as pl

K = 64
ROWS = 128
COLS = 32768
BLOCK_ROWS = 8


def _topk_mask_block(x_ref, o_ref):
    x = x_ref[...]
    b = lax.bitcast_convert_type(x, jnp.int32)
    s = jnp.where(b < 0, b ^ jnp.int32(0x7FFFFFFF), b)

    cnt_nonneg = jnp.sum((s >= 0).astype(jnp.int32), axis=1, keepdims=True)
    t0 = jnp.where(cnt_nonneg >= K, jnp.int32(0), jnp.int32(-0x80000000))

    def value_bit_step(i, t):
        bit = jnp.int32(1) << (jnp.int32(30) - i)
        cand = t | bit
        cnt = jnp.sum((s >= cand).astype(jnp.int32), axis=1, keepdims=True)
        return jnp.where(cnt >= K, cand, t)

    t = lax.fori_loop(0, 31, value_bit_step, t0)

    cnt_gt = jnp.sum((s > t).astype(jnp.int32), axis=1, keepdims=True)
    need_eq = K - cnt_gt
    eq = s == t
    idx = lax.broadcasted_iota(jnp.int32, x.shape, 1)

    def index_bit_step(i, j):
        cand = j | (jnp.int32(1) << (jnp.int32(14) - i))
        cnt = jnp.sum((eq & (idx <= cand)).astype(jnp.int32), axis=1,
                      keepdims=True)
        return jnp.where(cnt <= need_eq, cand, j)

    j = lax.fori_loop(0, 15, index_bit_step, jnp.zeros_like(t))

    keep = (s > t) | (eq & (idx <= j))
    o_ref[...] = jnp.where(keep, x, jnp.float32(0.0))


@jax.jit
def kernel(x):
    return pl.pallas_call(
        _topk_mask_block,
        grid=(ROWS // BLOCK_ROWS,),
        in_specs=[pl.BlockSpec((BLOCK_ROWS, COLS), lambda i: (i, 0))],
        out_specs=pl.BlockSpec((BLOCK_ROWS, COLS), lambda i: (i, 0)),
        out_shape=jax.ShapeDtypeStruct((ROWS, COLS), jnp.float32),
    )(x)
